# Initial kernel scaffold; baseline (speedup 1.0000x reference)
import jax
import jax.numpy as jnp
from jax.experimental import pallas as pl


def kernel(feat_lr, guide_hr, sx_raw, sy_raw, sz_raw, sr_raw):
    raise NotImplementedError("write your pallas kernel here")


# speedup vs baseline: 2103.5623x; 2103.5623x over previous
import jax
import jax.numpy as jnp
import numpy as np
from jax.experimental import pallas as pl
from jax.experimental.pallas import tpu as pltpu

_HL = 16
_R_MAX = 2
_C = 32
_G = 3

_OFFS = [
    (dx, dy, dz)
    for dx in range(-_R_MAX, _R_MAX + 1)
    for dy in range(-_R_MAX, _R_MAX + 1)
    for dz in range(-_R_MAX, _R_MAX + 1)
    if dx * dx + dy * dy + dz * dz <= _R_MAX * _R_MAX
]
_D2 = [dx * dx + dy * dy + dz * dz for (dx, dy, dz) in _OFFS]
_NOFF = len(_OFFS)


def _dist2_tables():
    k = np.arange(_HL)
    base = np.zeros((2, 2 * _R_MAX + 1, _HL), np.float32)
    for p in (0, 1):
        for di, d in enumerate(range(-_R_MAX, _R_MAX + 1)):
            ui = np.clip(k + d, 0, _HL - 1)
            base[p, di] = (2 * k + p - (2 * ui + 0.5)) ** 2
    dx2r = np.zeros((_NOFF, 8, _HL), np.float32)
    dy2f = np.zeros((_NOFF, 8, _HL * _HL), np.float32)
    dz2f = np.zeros((_NOFF, 8, _HL * _HL), np.float32)
    j = np.arange(_HL * _HL)
    for n, (dx, dy, dz) in enumerate(_OFFS):
        for p in range(8):
            px, py, pz = (p >> 2) & 1, (p >> 1) & 1, p & 1
            dx2r[n, p] = base[px, dx + _R_MAX]
            dy2f[n, p] = base[py, dy + _R_MAX][j // _HL]
            dz2f[n, p] = base[pz, dz + _R_MAX][j % _HL]
    return dx2r, dy2f, dz2f


_DX2R, _DY2F, _DZ2F = _dist2_tables()


def _kz(shape):
    j = jax.lax.broadcasted_iota(jnp.int32, shape, len(shape) - 1)
    return j & 15


def _zshift_clamp(a, dz):
    if dz == 0:
        return a
    kz = _kz(a.shape)
    if dz > 0:
        s1 = jnp.concatenate([a[..., 1:], a[..., -1:]], axis=-1)
        if dz == 1:
            return jnp.where(kz == 15, a, s1)
        s2 = jnp.concatenate([a[..., 2:], a[..., -2:]], axis=-1)
        return jnp.where(kz == 15, a, jnp.where(kz == 14, s1, s2))
    s1 = jnp.concatenate([a[..., :1], a[..., :-1]], axis=-1)
    if dz == -1:
        return jnp.where(kz == 0, a, s1)
    s2 = jnp.concatenate([a[..., :2], a[..., :-2]], axis=-1)
    return jnp.where(kz == 0, a, jnp.where(kz == 1, s1, s2))


def _zshift_zero(a, dz):
    kz = _kz(a.shape)
    if dz == 1:
        s1 = jnp.concatenate([a[..., 1:], a[..., -1:]], axis=-1)
        return jnp.where(kz == 15, 0.0, s1)
    s1 = jnp.concatenate([a[..., :1], a[..., :-1]], axis=-1)
    return jnp.where(kz == 0, 0.0, s1)


def _yshift_clamp(a, dy):
    if dy == 0:
        return a
    s = 16 * dy
    if dy > 0:
        return jnp.concatenate([a[..., s:]] + [a[..., 240:]] * dy, axis=-1)
    return jnp.concatenate([a[..., :16]] * (-dy) + [a[..., :256 + s]], axis=-1)


def _yshift_zero(a, dy):
    zero = jnp.zeros(a.shape[:-1] + (16,), a.dtype)
    if dy == 1:
        return jnp.concatenate([a[..., 16:], zero], axis=-1)
    return jnp.concatenate([zero, a[..., :-16]], axis=-1)


def _xshift_clamp(a, dx):
    if dx == 0:
        return a
    if dx > 0:
        return jnp.concatenate(
            [a[..., dx:, :]] + [a[..., -1:, :]] * dx, axis=-2)
    return jnp.concatenate([a[..., :1, :]] * (-dx) + [a[..., :dx, :]], axis=-2)


def _xshift_zero(a, dx):
    zero = jnp.zeros(a.shape[:-2] + (1, a.shape[-1]), a.dtype)
    if dx == 1:
        return jnp.concatenate([a[..., 1:, :], zero], axis=-2)
    return jnp.concatenate([zero, a[..., :-1, :]], axis=-2)


def _kernel_body(feat_ref, gpar_ref, sraw_ref, dx2_ref, dy2_ref, dz2_ref,
                 out_ref, vs_ref, vf_ref, w_ref, thr_ref):
    sx = jnp.exp(sraw_ref[0])
    sy = jnp.exp(sraw_ref[1])
    sz = jnp.exp(sraw_ref[2])
    sr = jnp.exp(sraw_ref[3])
    asx = 1.0 / (2.0 * jnp.maximum(sx, 1e-6) ** 2)
    asy = 1.0 / (2.0 * jnp.maximum(sy, 1e-6) ** 2)
    asz = 1.0 / (2.0 * jnp.maximum(sz, 1e-6) ** 2)
    iwr = 1.0 / (2.0 * jnp.maximum(sr, 1e-6) ** 2 + 1e-8)
    seff = jnp.maximum(sx, jnp.maximum(sy, sz))

    def upx(a, p):
        return 0.75 * a + 0.25 * _xshift_clamp(a, -1 if p == 0 else 1)

    def upy(a, p):
        return 0.75 * a + 0.25 * _yshift_clamp(a, -1 if p == 0 else 1)

    def upz(a, p):
        return 0.75 * a + 0.25 * _zshift_clamp(a, -1 if p == 0 else 1)

    for p in range(8):
        px, py, pz = (p >> 2) & 1, (p >> 1) & 1, p & 1
        thr_ref[p] = upz(upy(upx(seff, px), py), pz)

    shp3 = (_G, 16, 256)
    row = jax.lax.broadcasted_iota(jnp.int32, shp3, 1)
    lane = jax.lax.broadcasted_iota(jnp.int32, shp3, 2)
    ky3, kz3 = lane >> 4, lane & 15

    def renorm(q):
        return jnp.where((q == 0) | (q == 15), 1.0 / 0.875, 1.0)

    def down(a0, a1, sh_zero, q):
        s = (0.375 * (a0 + a1)
             + 0.125 * (sh_zero(a1, -1) + sh_zero(a0, 1)))
        return s * renorm(q)

    gp = [gpar_ref[p] for p in range(8)]
    dnx = {}
    for py in (0, 1):
        for pz in (0, 1):
            dnx[(py, pz)] = down(gp[2 * py + pz], gp[4 + 2 * py + pz],
                                 _xshift_zero, row)
    dny = {}
    for pz in (0, 1):
        dny[pz] = down(dnx[(0, pz)], dnx[(1, pz)], _yshift_zero, ky3)
    glr = down(dny[0], dny[1], _zshift_zero, kz3)

    def xpad(a):
        return jnp.concatenate(
            [a[..., :1, :]] * 2 + [a] + [a[..., -1:, :]] * 2, axis=-2)

    small_maps = [asx, asy, asz, iwr, glr[0], glr[1], glr[2]]
    for dzi, dz in enumerate(range(-_R_MAX, _R_MAX + 1)):
        for mi, mp in enumerate(small_maps):
            vs_ref[dzi, mi] = xpad(_zshift_clamp(mp, dz))
        vf_ref[dzi] = xpad(_zshift_clamp(feat_ref[...], dz))

    def take_s(dzi, mi, dx, dy):
        return _yshift_clamp(vs_ref[dzi, mi, 2 + dx:18 + dx, :], dy)

    m = None
    for n, (dx, dy, dz) in enumerate(_OFFS):
        dzi = dz + _R_MAX
        w = -(dx2_ref[n][:, :, None] * take_s(dzi, 0, dx, dy)[None]
              + dy2_ref[n][:, None, :] * take_s(dzi, 1, dx, dy)[None]
              + dz2_ref[n][:, None, :] * take_s(dzi, 2, dx, dy)[None])
        diff2 = None
        for g in range(_G):
            d = gpar_ref[:, g] - take_s(dzi, 4 + g, dx, dy)[None]
            diff2 = d * d if diff2 is None else diff2 + d * d
        w = w - diff2 * take_s(dzi, 3, dx, dy)[None]
        if _D2[n] > 1:
            w = jnp.where(thr_ref[...] > 0.5, w, -1e9)
        w_ref[n] = w
        m = w if m is None else jnp.maximum(m, w)

    den = None
    for n in range(_NOFF):
        w = jnp.exp(w_ref[n] - m)
        w_ref[n] = w
        den = w if den is None else den + w
    inv = 1.0 / jnp.maximum(den, 1e-8)

    CB = 4
    for c0 in range(0, _C, CB):
        acc = None
        for n, (dx, dy, dz) in enumerate(_OFFS):
            dzi = dz + _R_MAX
            f = _yshift_clamp(
                vf_ref[dzi, c0:c0 + CB, 2 + dx:18 + dx, :], dy)
            contrib = w_ref[n][:, None] * f[None]
            acc = contrib if acc is None else acc + contrib
        out_ref[:, c0:c0 + CB] = acc * inv[:, None]


def kernel(feat_lr, guide_hr, sx_raw, sy_raw, sz_raw, sr_raw):
    feat = feat_lr[0].reshape(_C, 16, 256)
    g = guide_hr[0]
    gpar = (g.reshape(_G, 16, 2, 16, 2, 16, 2)
            .transpose(2, 4, 6, 0, 1, 3, 5)
            .reshape(8, _G, 16, 256))
    sraw = jnp.concatenate(
        [sx_raw[0], sy_raw[0], sz_raw[0], sr_raw[0]], axis=0
    ).reshape(4, 16, 256)

    out = pl.pallas_call(
        _kernel_body,
        out_shape=jax.ShapeDtypeStruct((8, _C, 16, 256), jnp.float32),
        scratch_shapes=[
            pltpu.VMEM((5, 7, 20, 256), jnp.float32),
            pltpu.VMEM((5, _C, 20, 256), jnp.float32),
            pltpu.VMEM((_NOFF, 8, 16, 256), jnp.float32),
            pltpu.VMEM((8, 16, 256), jnp.float32),
        ],
    )(feat, gpar, sraw,
      jnp.asarray(_DX2R), jnp.asarray(_DY2F), jnp.asarray(_DZ2F))

    out = (out.reshape(2, 2, 2, _C, 16, 16, 16)
           .transpose(3, 4, 0, 5, 1, 6, 2)
           .reshape(_C, 32, 32, 32))
    return out[None]


# speedup vs baseline: 4189.7466x; 1.9917x over previous
import jax
import jax.numpy as jnp
import numpy as np
from jax.experimental import pallas as pl
from jax.experimental.pallas import tpu as pltpu

_HL = 16
_R_MAX = 2
_C = 32
_G = 3
_NL = 1024

_OFFS = [
    (dx, dy, dz)
    for dx in range(-_R_MAX, _R_MAX + 1)
    for dy in range(-_R_MAX, _R_MAX + 1)
    for dz in range(-_R_MAX, _R_MAX + 1)
    if dx * dx + dy * dy + dz * dz <= _R_MAX * _R_MAX
]
_D2 = [dx * dx + dy * dy + dz * dz for (dx, dy, dz) in _OFFS]
_NOFF = len(_OFFS)


def _dist2_tables():
    k = np.arange(_HL)
    base = np.zeros((2, 2 * _R_MAX + 1, _HL), np.float32)
    for p in (0, 1):
        for di, d in enumerate(range(-_R_MAX, _R_MAX + 1)):
            ui = np.clip(k + d, 0, _HL - 1)
            base[p, di] = (2 * k + p - (2 * ui + 0.5)) ** 2
    J = np.arange(_NL)
    kyJ, pyJ = J >> 6, (J >> 5) & 1
    kzJ, pzJ = (J >> 1) & 15, J & 1
    dx2r = np.zeros((_NOFF, 2, _HL), np.float32)
    dy2f = np.zeros((_NOFF, _NL), np.float32)
    dz2f = np.zeros((_NOFF, _NL), np.float32)
    for n, (dx, dy, dz) in enumerate(_OFFS):
        dx2r[n] = base[:, dx + _R_MAX, :]
        dy2f[n] = base[pyJ, dy + _R_MAX, kyJ]
        dz2f[n] = base[pzJ, dz + _R_MAX, kzJ]
    return dx2r, dy2f, dz2f


_DX2R, _DY2F, _DZ2F = _dist2_tables()


def _lane_iota(shape):
    return jax.lax.broadcasted_iota(jnp.int32, shape, len(shape) - 1)


def _sh_lane(a, t):
    if t == 0:
        return a
    zero = jnp.zeros(a.shape[:-1] + (abs(t),), a.dtype)
    if t > 0:
        return jnp.concatenate([a[..., t:], zero], axis=-1)
    return jnp.concatenate([zero, a[..., :t]], axis=-1)


def _zshift_clamp(a, dz):
    if dz == 0:
        return a
    kz = (_lane_iota(a.shape) >> 1) & 15
    if dz > 0:
        s1 = jnp.concatenate([a[..., 2:], a[..., -2:]], axis=-1)
        if dz == 1:
            return jnp.where(kz == 15, a, s1)
        s2 = jnp.concatenate([a[..., 4:], a[..., -4:]], axis=-1)
        return jnp.where(kz == 15, a, jnp.where(kz == 14, s1, s2))
    s1 = jnp.concatenate([a[..., :2], a[..., :-2]], axis=-1)
    if dz == -1:
        return jnp.where(kz == 0, a, s1)
    s2 = jnp.concatenate([a[..., :4], a[..., :-4]], axis=-1)
    return jnp.where(kz == 0, a, jnp.where(kz == 1, s1, s2))


def _yshift_clamp(a, dy):
    if dy == 0:
        return a
    s = 64 * dy
    if dy > 0:
        return jnp.concatenate([a[..., s:]] + [a[..., _NL - 64:]] * dy, axis=-1)
    return jnp.concatenate([a[..., :64]] * (-dy) + [a[..., :_NL + s]], axis=-1)


def _xshift_clamp(a, dx):
    if dx == 0:
        return a
    if dx > 0:
        return jnp.concatenate(
            [a[..., dx:, :]] + [a[..., -1:, :]] * dx, axis=-2)
    return jnp.concatenate([a[..., :1, :]] * (-dx) + [a[..., :dx, :]], axis=-2)


def _xshift_zero(a, dx):
    zero = jnp.zeros(a.shape[:-2] + (1, a.shape[-1]), a.dtype)
    if dx == 1:
        return jnp.concatenate([a[..., 1:, :], zero], axis=-2)
    return jnp.concatenate([zero, a[..., :-1, :]], axis=-2)


def _kernel_body(feat_ref, ghr_ref, sraw_ref, dx2_ref, dy2_ref, dz2_ref,
                 out_ref, vs_ref, vf_ref, w_ref, thr_ref):
    shp = (16, _NL)
    lane = _lane_iota(shp)
    ky, py = lane >> 6, (lane >> 5) & 1
    kz, pz = (lane >> 1) & 15, lane & 1
    Z = lane & 31
    rowi = jax.lax.broadcasted_iota(jnp.int32, shp, 0)

    sx = jnp.exp(sraw_ref[0])
    sy = jnp.exp(sraw_ref[1])
    sz = jnp.exp(sraw_ref[2])
    sr = jnp.exp(sraw_ref[3])
    asx = 1.0 / (2.0 * jnp.maximum(sx, 1e-6) ** 2)
    asy = 1.0 / (2.0 * jnp.maximum(sy, 1e-6) ** 2)
    asz = 1.0 / (2.0 * jnp.maximum(sz, 1e-6) ** 2)
    iwr = 1.0 / (2.0 * jnp.maximum(sr, 1e-6) ** 2 + 1e-8)
    seff = jnp.maximum(sx, jnp.maximum(sy, sz))

    uy = 0.75 * seff + 0.25 * jnp.where(
        py == 0, _yshift_clamp(seff, -1), _yshift_clamp(seff, 1))
    uyz = 0.75 * uy + 0.25 * jnp.where(
        pz == 0, _zshift_clamp(uy, -1), _zshift_clamp(uy, 1))
    for px in range(2):
        thr_ref[px] = (0.75 * uyz
                       + 0.25 * _xshift_clamp(uyz, -1 if px == 0 else 1))

    def renorm(q):
        return jnp.where((q == 0) | (q == 15), 1.0 / 0.875, 1.0)

    g0 = ghr_ref[0]
    g1 = ghr_ref[1]
    gx = (0.375 * (g0 + g1)
          + 0.125 * (_xshift_zero(g1, -1) + _xshift_zero(g0, 1))
          ) * renorm(rowi)[None]

    def shY(a, o):
        return _sh_lane(a, 32 * o)

    gxy = jnp.where(
        py == 0,
        0.125 * shY(gx, -1) + 0.375 * gx + 0.375 * shY(gx, 1)
        + 0.125 * shY(gx, 2),
        0.125 * shY(gx, -2) + 0.375 * shY(gx, -1) + 0.375 * gx
        + 0.125 * shY(gx, 1)) * renorm(ky)[None]

    def shZ(a, o):
        return jnp.where((Z + o >= 0) & (Z + o <= 31), _sh_lane(a, o), 0.0)

    glr = jnp.where(
        pz == 0,
        0.125 * shZ(gxy, -1) + 0.375 * gxy + 0.375 * shZ(gxy, 1)
        + 0.125 * shZ(gxy, 2),
        0.125 * shZ(gxy, -2) + 0.375 * shZ(gxy, -1) + 0.375 * gxy
        + 0.125 * shZ(gxy, 1)) * renorm(kz)[None]

    def xpad(a):
        return jnp.concatenate(
            [a[..., :1, :]] * 2 + [a] + [a[..., -1:, :]] * 2, axis=-2)

    small_maps = [asx, asy, asz, iwr, glr[0], glr[1], glr[2]]
    for dzi, dz in enumerate(range(-_R_MAX, _R_MAX + 1)):
        for mi, mp in enumerate(small_maps):
            vs_ref[dzi, mi] = xpad(_zshift_clamp(mp, dz))
        vf_ref[dzi] = xpad(_zshift_clamp(feat_ref[...], dz))

    def take_s(dzi, mi, dx, dy):
        return _yshift_clamp(vs_ref[dzi, mi, 2 + dx:18 + dx, :], dy)

    m = None
    for n, (dx, dy, dz) in enumerate(_OFFS):
        dzi = dz + _R_MAX
        w = -(dx2_ref[n][:, :, None] * take_s(dzi, 0, dx, dy)[None]
              + dy2_ref[n][None, None, :] * take_s(dzi, 1, dx, dy)[None]
              + dz2_ref[n][None, None, :] * take_s(dzi, 2, dx, dy)[None])
        diff2 = None
        for g in range(_G):
            d = ghr_ref[:, g] - take_s(dzi, 4 + g, dx, dy)[None]
            diff2 = d * d if diff2 is None else diff2 + d * d
        w = w - diff2 * take_s(dzi, 3, dx, dy)[None]
        if _D2[n] > 1:
            w = jnp.where(thr_ref[...] > 0.5, w, -1e9)
        w_ref[n] = w
        m = w if m is None else jnp.maximum(m, w)

    den = None
    for n in range(_NOFF):
        w = jnp.exp(w_ref[n] - m)
        w_ref[n] = w
        den = w if den is None else den + w
    inv = 1.0 / jnp.maximum(den, 1e-8)

    CB = 4
    for c0 in range(0, _C, CB):
        acc = None
        for n, (dx, dy, dz) in enumerate(_OFFS):
            dzi = dz + _R_MAX
            f = _yshift_clamp(
                vf_ref[dzi, c0:c0 + CB, 2 + dx:18 + dx, :], dy)
            contrib = w_ref[n][:, None] * f[None]
            acc = contrib if acc is None else acc + contrib
        out_ref[:, c0:c0 + CB] = acc * inv[:, None]


def kernel(feat_lr, guide_hr, sx_raw, sy_raw, sz_raw, sr_raw):
    def dup(a):
        b = a[..., :, None, :, None]
        b = jnp.broadcast_to(b, a.shape[:-2] + (16, 2, 16, 2))
        return b.reshape(a.shape[:-3] + (16, _NL))

    feat = dup(feat_lr[0])
    ghr = (guide_hr[0].reshape(_G, 16, 2, _NL)
           .transpose(2, 0, 1, 3))
    sraw = dup(jnp.concatenate(
        [sx_raw[0], sy_raw[0], sz_raw[0], sr_raw[0]], axis=0))

    out = pl.pallas_call(
        _kernel_body,
        out_shape=jax.ShapeDtypeStruct((2, _C, 16, _NL), jnp.float32),
        scratch_shapes=[
            pltpu.VMEM((5, 7, 20, _NL), jnp.float32),
            pltpu.VMEM((5, _C, 20, _NL), jnp.float32),
            pltpu.VMEM((_NOFF, 2, 16, _NL), jnp.float32),
            pltpu.VMEM((2, 16, _NL), jnp.float32),
        ],
    )(feat, ghr, sraw,
      jnp.asarray(_DX2R), jnp.asarray(_DY2F), jnp.asarray(_DZ2F))

    out = out.transpose(1, 2, 0, 3).reshape(_C, 32, 32, 32)
    return out[None]


# speedup vs baseline: 4292.4686x; 1.0245x over previous
import jax
import jax.numpy as jnp
import numpy as np
from jax.experimental import pallas as pl
from jax.experimental.pallas import tpu as pltpu

_HL = 16
_R_MAX = 2
_C = 32
_G = 3
_NL = 1024

_OFFS = [
    (dx, dy, dz)
    for dx in range(-_R_MAX, _R_MAX + 1)
    for dy in range(-_R_MAX, _R_MAX + 1)
    for dz in range(-_R_MAX, _R_MAX + 1)
    if dx * dx + dy * dy + dz * dz <= _R_MAX * _R_MAX
]
_D2 = [dx * dx + dy * dy + dz * dz for (dx, dy, dz) in _OFFS]
_NOFF = len(_OFFS)

_GROUPS = []
for _dy in range(-_R_MAX, _R_MAX + 1):
    for _dz in range(-_R_MAX, _R_MAX + 1):
        dxs = [(dx, n) for n, (dx, dy, dz) in enumerate(_OFFS)
               if dy == _dy and dz == _dz]
        if dxs:
            _GROUPS.append((_dy, _dz, dxs))


def _dist2_tables():
    k = np.arange(_HL)
    base = np.zeros((2, 2 * _R_MAX + 1, _HL), np.float32)
    for p in (0, 1):
        for di, d in enumerate(range(-_R_MAX, _R_MAX + 1)):
            ui = np.clip(k + d, 0, _HL - 1)
            base[p, di] = (2 * k + p - (2 * ui + 0.5)) ** 2
    J = np.arange(_NL)
    kyJ, pyJ = J >> 6, (J >> 5) & 1
    kzJ, pzJ = (J >> 1) & 15, J & 1
    dx2r = np.zeros((_NOFF, 2, _HL), np.float32)
    dy2f = np.zeros((_NOFF, _NL), np.float32)
    dz2f = np.zeros((_NOFF, _NL), np.float32)
    for n, (dx, dy, dz) in enumerate(_OFFS):
        dx2r[n] = base[:, dx + _R_MAX, :]
        dy2f[n] = base[pyJ, dy + _R_MAX, kyJ]
        dz2f[n] = base[pzJ, dz + _R_MAX, kzJ]
    return dx2r, dy2f, dz2f


_DX2R, _DY2F, _DZ2F = _dist2_tables()


def _lane_iota(shape):
    return jax.lax.broadcasted_iota(jnp.int32, shape, len(shape) - 1)


def _sh_lane(a, t):
    if t == 0:
        return a
    zero = jnp.zeros(a.shape[:-1] + (abs(t),), a.dtype)
    if t > 0:
        return jnp.concatenate([a[..., t:], zero], axis=-1)
    return jnp.concatenate([zero, a[..., :t]], axis=-1)


def _zshift_clamp(a, dz):
    if dz == 0:
        return a
    kz = (_lane_iota(a.shape) >> 1) & 15
    if dz > 0:
        s1 = jnp.concatenate([a[..., 2:], a[..., -2:]], axis=-1)
        if dz == 1:
            return jnp.where(kz == 15, a, s1)
        s2 = jnp.concatenate([a[..., 4:], a[..., -4:]], axis=-1)
        return jnp.where(kz == 15, a, jnp.where(kz == 14, s1, s2))
    s1 = jnp.concatenate([a[..., :2], a[..., :-2]], axis=-1)
    if dz == -1:
        return jnp.where(kz == 0, a, s1)
    s2 = jnp.concatenate([a[..., :4], a[..., :-4]], axis=-1)
    return jnp.where(kz == 0, a, jnp.where(kz == 1, s1, s2))


def _yshift_clamp(a, dy):
    if dy == 0:
        return a
    s = 64 * dy
    if dy > 0:
        return jnp.concatenate([a[..., s:]] + [a[..., _NL - 64:]] * dy, axis=-1)
    return jnp.concatenate([a[..., :64]] * (-dy) + [a[..., :_NL + s]], axis=-1)


def _xshift_clamp(a, dx):
    if dx == 0:
        return a
    if dx > 0:
        return jnp.concatenate(
            [a[..., dx:, :]] + [a[..., -1:, :]] * dx, axis=-2)
    return jnp.concatenate([a[..., :1, :]] * (-dx) + [a[..., :dx, :]], axis=-2)


def _xshift_zero(a, dx):
    zero = jnp.zeros(a.shape[:-2] + (1, a.shape[-1]), a.dtype)
    if dx == 1:
        return jnp.concatenate([a[..., 1:, :], zero], axis=-2)
    return jnp.concatenate([zero, a[..., :-1, :]], axis=-2)


def _kernel_body(feat_ref, ghr_ref, sraw_ref, dx2_ref, dy2_ref, dz2_ref,
                 out_ref, vs_ref, vf_ref, w_ref, thr_ref):
    shp = (16, _NL)
    lane = _lane_iota(shp)
    ky, py = lane >> 6, (lane >> 5) & 1
    kz, pz = (lane >> 1) & 15, lane & 1
    Z = lane & 31
    rowi = jax.lax.broadcasted_iota(jnp.int32, shp, 0)

    sx = jnp.exp(sraw_ref[0])
    sy = jnp.exp(sraw_ref[1])
    sz = jnp.exp(sraw_ref[2])
    sr = jnp.exp(sraw_ref[3])
    asx = 1.0 / (2.0 * jnp.maximum(sx, 1e-6) ** 2)
    asy = 1.0 / (2.0 * jnp.maximum(sy, 1e-6) ** 2)
    asz = 1.0 / (2.0 * jnp.maximum(sz, 1e-6) ** 2)
    iwr = 1.0 / (2.0 * jnp.maximum(sr, 1e-6) ** 2 + 1e-8)
    seff = jnp.maximum(sx, jnp.maximum(sy, sz))

    uy = 0.75 * seff + 0.25 * jnp.where(
        py == 0, _yshift_clamp(seff, -1), _yshift_clamp(seff, 1))
    uyz = 0.75 * uy + 0.25 * jnp.where(
        pz == 0, _zshift_clamp(uy, -1), _zshift_clamp(uy, 1))
    for px in range(2):
        thr_ref[px] = (0.75 * uyz
                       + 0.25 * _xshift_clamp(uyz, -1 if px == 0 else 1))

    def renorm(q):
        return jnp.where((q == 0) | (q == 15), 1.0 / 0.875, 1.0)

    g0 = ghr_ref[0]
    g1 = ghr_ref[1]
    gx = (0.375 * (g0 + g1)
          + 0.125 * (_xshift_zero(g1, -1) + _xshift_zero(g0, 1))
          ) * renorm(rowi)[None]

    def shY(a, o):
        return _sh_lane(a, 32 * o)

    gxy = jnp.where(
        py == 0,
        0.125 * shY(gx, -1) + 0.375 * gx + 0.375 * shY(gx, 1)
        + 0.125 * shY(gx, 2),
        0.125 * shY(gx, -2) + 0.375 * shY(gx, -1) + 0.375 * gx
        + 0.125 * shY(gx, 1)) * renorm(ky)[None]

    def shZ(a, o):
        return jnp.where((Z + o >= 0) & (Z + o <= 31), _sh_lane(a, o), 0.0)

    glr = jnp.where(
        pz == 0,
        0.125 * shZ(gxy, -1) + 0.375 * gxy + 0.375 * shZ(gxy, 1)
        + 0.125 * shZ(gxy, 2),
        0.125 * shZ(gxy, -2) + 0.375 * shZ(gxy, -1) + 0.375 * gxy
        + 0.125 * shZ(gxy, 1)) * renorm(kz)[None]

    def xpad(a):
        return jnp.concatenate(
            [a[..., :1, :]] * 2 + [a] + [a[..., -1:, :]] * 2, axis=-2)

    small_maps = [asx, asy, asz, iwr, glr[0], glr[1], glr[2]]
    for dzi, dz in enumerate(range(-_R_MAX, _R_MAX + 1)):
        for mi, mp in enumerate(small_maps):
            vs_ref[dzi, mi] = xpad(_zshift_clamp(mp, dz))
        vf_ref[dzi] = xpad(_zshift_clamp(feat_ref[...], dz))

    m = None
    for gy, gz, dxs in _GROUPS:
        dzi = gz + _R_MAX
        ms = [_yshift_clamp(vs_ref[dzi, mi], gy) for mi in range(7)]
        for dx, n in dxs:
            sl = slice(2 + dx, 18 + dx)
            w = -(dx2_ref[n][:, :, None] * ms[0][sl][None]
                  + dy2_ref[n][None, None, :] * ms[1][sl][None]
                  + dz2_ref[n][None, None, :] * ms[2][sl][None])
            diff2 = None
            for g in range(_G):
                d = ghr_ref[:, g] - ms[4 + g][sl][None]
                diff2 = d * d if diff2 is None else diff2 + d * d
            w = w - diff2 * ms[3][sl][None]
            if _D2[n] > 1:
                w = jnp.where(thr_ref[...] > 0.5, w, -1e9)
            w_ref[n] = w
            m = w if m is None else jnp.maximum(m, w)

    den = None
    for n in range(_NOFF):
        w = jnp.exp(w_ref[n] - m)
        w_ref[n] = w
        den = w if den is None else den + w
    inv = 1.0 / jnp.maximum(den, 1e-8)

    CB = 4
    for c0 in range(0, _C, CB):
        acc = None
        for gy, gz, dxs in _GROUPS:
            dzi = gz + _R_MAX
            fyz = _yshift_clamp(vf_ref[dzi, c0:c0 + CB], gy)
            for dx, n in dxs:
                f = fyz[:, 2 + dx:18 + dx, :]
                contrib = w_ref[n][:, None] * f[None]
                acc = contrib if acc is None else acc + contrib
        out_ref[:, c0:c0 + CB] = acc * inv[:, None]


def kernel(feat_lr, guide_hr, sx_raw, sy_raw, sz_raw, sr_raw):
    def dup(a):
        b = a[..., :, None, :, None]
        b = jnp.broadcast_to(b, a.shape[:-2] + (16, 2, 16, 2))
        return b.reshape(a.shape[:-3] + (16, _NL))

    feat = dup(feat_lr[0])
    ghr = (guide_hr[0].reshape(_G, 16, 2, _NL)
           .transpose(2, 0, 1, 3))
    sraw = dup(jnp.concatenate(
        [sx_raw[0], sy_raw[0], sz_raw[0], sr_raw[0]], axis=0))

    out = pl.pallas_call(
        _kernel_body,
        out_shape=jax.ShapeDtypeStruct((2, _C, 16, _NL), jnp.float32),
        scratch_shapes=[
            pltpu.VMEM((5, 7, 20, _NL), jnp.float32),
            pltpu.VMEM((5, _C, 20, _NL), jnp.float32),
            pltpu.VMEM((_NOFF, 2, 16, _NL), jnp.float32),
            pltpu.VMEM((2, 16, _NL), jnp.float32),
        ],
    )(feat, ghr, sraw,
      jnp.asarray(_DX2R), jnp.asarray(_DY2F), jnp.asarray(_DZ2F))

    out = out.transpose(1, 2, 0, 3).reshape(_C, 32, 32, 32)
    return out[None]
